# Initial kernel scaffold; baseline (speedup 1.0000x reference)
import jax
import jax.numpy as jnp
from jax.experimental import pallas as pl


def kernel(coarse_codebook, coarse_indices, residual_cb_0, residual_cb_1, residual_idx_0, residual_idx_1):
    raise NotImplementedError("write your pallas kernel here")


# speedup vs baseline: 10.0295x; 10.0295x over previous
import functools

import jax
import jax.numpy as jnp
from jax import lax
from jax.experimental import pallas as pl
from jax.experimental.pallas import tpu as pltpu
from jax.experimental.pallas import tpu_sc as plsc

K_C = 8192
D = 256
K_R = 16
N = 32768


_BK = 512


def _onehot_argmax(scores):
    m = jnp.max(scores, axis=1, keepdims=True)
    eq = scores == m
    col = lax.broadcasted_iota(jnp.int32, scores.shape, 1)
    first = jnp.min(jnp.where(eq, col, K_R), axis=1, keepdims=True)
    return (col == first).astype(jnp.float32)


def _fuse_body(cb_ref, r0_ref, r1_ref, rcb0_ref, rcb1_ref, out_ref):
    oh0 = _onehot_argmax(r0_ref[...])
    oh1 = _onehot_argmax(r1_ref[...])
    e0 = jnp.dot(oh0, rcb0_ref[...], preferred_element_type=jnp.float32)
    e1 = jnp.dot(oh1, rcb1_ref[...], preferred_element_type=jnp.float32)
    out_ref[...] = cb_ref[...] + e0 + e1


def _build_fused(codebook, ridx0, ridx1, cb0, cb1):
    return pl.pallas_call(
        _fuse_body,
        grid=(K_C // _BK,),
        in_specs=[
            pl.BlockSpec((_BK, D), lambda i: (i, 0)),
            pl.BlockSpec((_BK, K_R), lambda i: (i, 0)),
            pl.BlockSpec((_BK, K_R), lambda i: (i, 0)),
            pl.BlockSpec((K_R, D), lambda i: (0, 0)),
            pl.BlockSpec((K_R, D), lambda i: (0, 0)),
        ],
        out_specs=pl.BlockSpec((_BK, D), lambda i: (i, 0)),
        out_shape=jax.ShapeDtypeStruct((K_C, D), jnp.float32),
    )(codebook, ridx0, ridx1, cb0, cb1)


_NC = 2
_NS = 16
_NW = _NC * _NS
_BPW = N // _NW
_CHUNK = 128
_NCHUNK = _BPW // _CHUNK


@functools.lru_cache(maxsize=None)
def _make_gather_kernel():
    @functools.partial(
        pl.kernel,
        out_type=jax.ShapeDtypeStruct((N, D), jnp.float32),
        mesh=plsc.VectorSubcoreMesh(core_axis_name="c", subcore_axis_name="s"),
        scratch_types=[
            pltpu.VMEM((_NCHUNK, _CHUNK), jnp.int32),
            pltpu.VMEM((_CHUNK, D), jnp.float32),
            pltpu.SemaphoreType.DMA,
        ],
    )
    def _gather_kernel(table_hbm, idx_hbm, out_hbm, idx_v, buf, gsem):
        wid = lax.axis_index("s") * _NC + lax.axis_index("c")
        base = wid * _BPW
        pltpu.sync_copy(idx_hbm.at[pl.ds(wid * _NCHUNK, _NCHUNK)], idx_v)
        for c in range(_NCHUNK):
            pltpu.async_copy(table_hbm.at[idx_v.at[c]], buf, gsem).wait()
            pltpu.sync_copy(buf, out_hbm.at[pl.ds(base + c * _CHUNK, _CHUNK)])

    return _gather_kernel


def kernel(coarse_codebook, coarse_indices, residual_cb_0, residual_cb_1,
           residual_idx_0, residual_idx_1):
    fused = _build_fused(coarse_codebook, residual_idx_0, residual_idx_1,
                         residual_cb_0, residual_cb_1)
    idx2d = coarse_indices.reshape(N // _CHUNK, _CHUNK)
    return _make_gather_kernel()(fused, idx2d)
